# baseline (speedup 1.0000x reference)
import jax
import jax.numpy as jnp
from jax.experimental import pallas as pl
from jax.experimental.pallas import tpu as pltpu

B, L, V = 4096, 20, 1000
C = V * L
CHUNK = 2560
ROWS = 64
K = 4
STEPS = B // ROWS


def _compute(t, base, buf_ref):
    for k in range(-(-C // CHUNK)):
        w = min(CHUNK, C - k * CHUNK)
        cols = base + (k * CHUNK)
        if w == CHUNK:
            buf_ref[:, k * CHUNK:(k + 1) * CHUNK] = (t == cols).astype(jnp.float32)
        else:
            buf_ref[:, k * CHUNK:k * CHUNK + w] = (
                (t[:, :w] == cols[:, :w]).astype(jnp.float32))


def _body(t_ref, out_ref, buf_ref, sem_ref):
    i = pl.program_id(0)
    t = t_ref[...]
    base = jax.lax.broadcasted_iota(jnp.int32, (ROWS, CHUNK), 1)
    slot = jax.lax.rem(i, K)
    for j in range(K):
        @pl.when(slot == j)
        def _(j=j):
            @pl.when(i >= K)
            def _():
                pltpu.make_async_copy(
                    buf_ref.at[j], out_ref.at[pl.ds(0, ROWS)], sem_ref.at[j]
                ).wait()
            _compute(t, base, buf_ref.at[j])
            pltpu.make_async_copy(
                buf_ref.at[j], out_ref.at[pl.ds(i * ROWS, ROWS)], sem_ref.at[j]
            ).start()

    @pl.when(i == STEPS - 1)
    def _():
        for j in range(K):
            pltpu.make_async_copy(
                buf_ref.at[j], out_ref.at[pl.ds(0, ROWS)], sem_ref.at[j]
            ).wait()


def kernel(inpt, train_flag):
    scaled = inpt.astype(jnp.int32) * L + jnp.arange(L, dtype=jnp.int32)
    tiled = jnp.tile(scaled, (1, CHUNK // L))
    out = pl.pallas_call(
        _body,
        grid=(STEPS,),
        in_specs=[pl.BlockSpec((ROWS, CHUNK), lambda i: (i, 0))],
        out_specs=pl.BlockSpec(memory_space=pl.ANY),
        out_shape=jax.ShapeDtypeStruct((B, C), jnp.float32),
        scratch_shapes=[
            pltpu.VMEM((K, ROWS, C), jnp.float32),
            pltpu.SemaphoreType.DMA((K,)),
        ],
    )(tiled)
    return out


# speedup vs baseline: 1.0016x; 1.0016x over previous
import jax
import jax.numpy as jnp
from jax.experimental import pallas as pl
from jax.experimental.pallas import tpu as pltpu

B, L, V = 4096, 20, 1000
C = V * L
CHUNK = 2560
ROWS = 64
K = 4
STEPS = B // ROWS


def _compute(t, base, buf_ref):
    for k in range(-(-C // CHUNK)):
        w = min(CHUNK, C - k * CHUNK)
        cols = base + (k * CHUNK)
        if w == CHUNK:
            buf_ref[:, k * CHUNK:(k + 1) * CHUNK] = (t == cols).astype(jnp.float32)
        else:
            buf_ref[:, k * CHUNK:k * CHUNK + w] = (
                (t[:, :w] == cols[:, :w]).astype(jnp.float32))


def _body(t_ref, out_ref, *bufs_and_sems):
    bufs = bufs_and_sems[:K]
    sem_ref = bufs_and_sems[K]
    i = pl.program_id(0)
    t = t_ref[...]
    base = jax.lax.broadcasted_iota(jnp.int32, (ROWS, CHUNK), 1)
    slot = jax.lax.rem(i, K)
    for j in range(K):
        @pl.when(slot == j)
        def _(j=j):
            @pl.when(i >= K)
            def _():
                pltpu.make_async_copy(
                    bufs[j], out_ref.at[pl.ds(0, ROWS)], sem_ref.at[j]
                ).wait()
            _compute(t, base, bufs[j])
            pltpu.make_async_copy(
                bufs[j], out_ref.at[pl.ds(i * ROWS, ROWS)], sem_ref.at[j]
            ).start()

    @pl.when(i == STEPS - 1)
    def _():
        for j in range(K):
            pltpu.make_async_copy(
                bufs[j], out_ref.at[pl.ds(0, ROWS)], sem_ref.at[j]
            ).wait()


def kernel(inpt, train_flag):
    scaled = inpt.astype(jnp.int32) * L + jnp.arange(L, dtype=jnp.int32)
    tiled = jnp.tile(scaled, (1, CHUNK // L))
    out = pl.pallas_call(
        _body,
        grid=(STEPS,),
        in_specs=[pl.BlockSpec((ROWS, CHUNK), lambda i: (i, 0))],
        out_specs=pl.BlockSpec(memory_space=pl.ANY),
        out_shape=jax.ShapeDtypeStruct((B, C), jnp.float32),
        scratch_shapes=(
            [pltpu.VMEM((ROWS, C), jnp.float32) for _ in range(K)]
            + [pltpu.SemaphoreType.DMA((K,))]
        ),
    )(tiled)
    return out


# speedup vs baseline: 1.2018x; 1.1998x over previous
import functools

import jax
import jax.numpy as jnp
from jax import lax
from jax.experimental import pallas as pl
from jax.experimental.pallas import tpu as pltpu
from jax.experimental.pallas import tpu_sc as plsc

B, L, V = 4096, 20, 1000
C = V * L
NW = 32
RPW = B // NW

_mesh = plsc.VectorSubcoreMesh(core_axis_name="c", subcore_axis_name="s")


@functools.partial(
    pl.kernel,
    out_type=jax.ShapeDtypeStruct((B, C), jnp.float32),
    mesh=_mesh,
    compiler_params=pltpu.CompilerParams(needs_layout_passes=False),
    scratch_types=[
        pltpu.VMEM((RPW * L,), jnp.int32),
        pltpu.VMEM((C,), jnp.float32),
        pltpu.VMEM((C,), jnp.float32),
        pltpu.SemaphoreType.DMA,
        pltpu.SemaphoreType.DMA,
    ],
)
def _sc_one_hot(scaled_hbm, out_hbm, idx_v, buf0, buf1, sem0, sem1):
    wid = lax.axis_index("s") * 2 + lax.axis_index("c")
    base = wid * RPW
    pltpu.sync_copy(scaled_hbm.at[pl.ds(base * L, RPW * L)], idx_v)

    lane = lax.iota(jnp.int32, 16)
    zeros = jnp.zeros((16,), jnp.float32)
    ones = jnp.full((16,), 1.0, jnp.float32)
    hi_mask = lane >= 12

    def memset(i, carry):
        buf0[pl.ds(i * 16, 16)] = zeros
        buf1[pl.ds(i * 16, 16)] = zeros
        return carry

    lax.fori_loop(0, C // 16, memset, 0)

    def scatter_row(buf, r, vals):
        off = r * L + lane
        g0 = plsc.load_gather(idx_v, [off])
        g1 = plsc.load_gather(idx_v, [off + 4])
        plsc.store_scatter(buf, [g0], vals)
        plsc.store_scatter(buf, [g1], vals, mask=hi_mask)

    def step(k, carry):
        r0 = 2 * k
        r1 = r0 + 1

        @pl.when(k > 0)
        def _():
            pltpu.make_async_copy(buf0, out_hbm.at[0], sem0).wait()
            scatter_row(buf0, r0 - 2, zeros)
        scatter_row(buf0, r0, ones)
        pltpu.async_copy(buf0, out_hbm.at[base + r0], sem0)

        @pl.when(k > 0)
        def _():
            pltpu.make_async_copy(buf1, out_hbm.at[0], sem1).wait()
            scatter_row(buf1, r1 - 2, zeros)
        scatter_row(buf1, r1, ones)
        pltpu.async_copy(buf1, out_hbm.at[base + r1], sem1)
        return carry

    lax.fori_loop(0, RPW // 2, step, 0)
    pltpu.make_async_copy(buf0, out_hbm.at[0], sem0).wait()
    pltpu.make_async_copy(buf1, out_hbm.at[0], sem1).wait()


def kernel(inpt, train_flag):
    scaled = inpt.astype(jnp.int32) * L + jnp.arange(L, dtype=jnp.int32)
    return _sc_one_hot(scaled.reshape(-1))


# speedup vs baseline: 1.2033x; 1.0013x over previous
import functools

import jax
import jax.numpy as jnp
from jax import lax
from jax.experimental import pallas as pl
from jax.experimental.pallas import tpu as pltpu
from jax.experimental.pallas import tpu_sc as plsc

B, L, V = 4096, 20, 1000
C = V * L
NW = 32
RPW = B // NW
RPB = 2
NBUF = 2
RPS = RPB * NBUF

_mesh = plsc.VectorSubcoreMesh(core_axis_name="c", subcore_axis_name="s")


@functools.partial(
    pl.kernel,
    out_type=jax.ShapeDtypeStruct((B, C), jnp.float32),
    mesh=_mesh,
    compiler_params=pltpu.CompilerParams(needs_layout_passes=False),
    scratch_types=[
        pltpu.VMEM((RPW * L,), jnp.int32),
        pltpu.VMEM((RPB, C), jnp.float32),
        pltpu.VMEM((RPB, C), jnp.float32),
        pltpu.SemaphoreType.DMA,
        pltpu.SemaphoreType.DMA,
    ],
)
def _sc_one_hot(scaled_hbm, out_hbm, idx_v, buf0, buf1, sem0, sem1):
    wid = lax.axis_index("s") * 2 + lax.axis_index("c")
    base = wid * RPW
    pltpu.sync_copy(scaled_hbm.at[pl.ds(base * L, RPW * L)], idx_v)

    lane = lax.iota(jnp.int32, 16)
    zeros = jnp.zeros((16,), jnp.float32)
    ones = jnp.full((16,), 1.0, jnp.float32)
    hi_mask = lane >= 12
    bufs = (buf0, buf1)
    sems = (sem0, sem1)

    def memset(i, carry):
        for jb in range(RPB):
            buf0[jb, pl.ds(i * 16, 16)] = zeros
            buf1[jb, pl.ds(i * 16, 16)] = zeros
        return carry

    lax.fori_loop(0, C // 16, memset, 0)

    def scatter_rows(buf, r, vals):
        for jb in range(RPB):
            off = (r + jb) * L + lane
            row = jnp.full((16,), jb, jnp.int32)
            g0 = plsc.load_gather(idx_v, [off])
            g1 = plsc.load_gather(idx_v, [off + 4])
            plsc.store_scatter(buf, [row, g0], vals)
            plsc.store_scatter(buf, [row, g1], vals, mask=hi_mask)

    def step(k, carry):
        for j in range(NBUF):
            r = k * RPS + j * RPB

            @pl.when(k > 0)
            def _(j=j, r=r):
                pltpu.make_async_copy(bufs[j], out_hbm.at[pl.ds(0, RPB)],
                                      sems[j]).wait()
                scatter_rows(bufs[j], r - RPS, zeros)
            scatter_rows(bufs[j], r, ones)
            pltpu.async_copy(bufs[j], out_hbm.at[pl.ds(base + r, RPB)], sems[j])
        return carry

    lax.fori_loop(0, RPW // RPS, step, 0)
    for j in range(NBUF):
        pltpu.make_async_copy(bufs[j], out_hbm.at[pl.ds(0, RPB)], sems[j]).wait()


def kernel(inpt, train_flag):
    scaled = inpt.astype(jnp.int32) * L + jnp.arange(L, dtype=jnp.int32)
    return _sc_one_hot(scaled.reshape(-1))
